# baseline (speedup 1.0000x reference)
import functools

import jax
import jax.numpy as jnp
from jax.experimental import pallas as pl
from jax.experimental.pallas import tpu as pltpu

B, T, V, U = 16, 2048, 512, 256
S = 2 * U + 1
S_PAD = 640
TAIL0 = 384
NEG = -1e30
T_BLK = 128
NT = T // T_BLK
NC = 2
BK = B // NC


def _fwd_kernel(ext_ref, skip_ref, lens_ref, lp_ref, num_ref, den_ref,
                e_scratch, alpha_ref):
    pid = pl.program_id(1)

    @pl.when(pid == 0)
    def _init():
        lane = jax.lax.broadcasted_iota(jnp.int32, (BK, S_PAD), 1)
        alpha_ref[...] = jnp.where(lane == 0, 0.0, NEG).astype(jnp.float32)
        num_ref[...] = jnp.zeros((BK, 1), jnp.float32)
        den_ref[...] = jnp.zeros((BK, 1), jnp.float32)

    iota_v = jax.lax.broadcasted_iota(jnp.int32, (V, S_PAD), 0)
    row_t = (jax.lax.broadcasted_iota(jnp.int32, (T_BLK, 1), 0)
             + pid * T_BLK)
    for b in range(BK):
        lp_b = lp_ref[b]
        onehot = (ext_ref[b:b + 1, :] == iota_v).astype(jnp.float32)
        e_scratch[:, b, :] = jnp.dot(lp_b, onehot,
                                     preferred_element_type=jnp.float32)
        m = jnp.max(lp_b, axis=1, keepdims=True)
        lse = m + jnp.log(jnp.sum(jnp.exp(lp_b - m), axis=1, keepdims=True))
        dpart = jnp.sum(jnp.where(row_t < lens_ref[b:b + 1, 0:1], lse, 0.0),
                        axis=0, keepdims=True)
        den_ref[b:b + 1, :] = den_ref[b:b + 1, :] + dpart

    skip_neg = skip_ref[...]
    lens = lens_ref[...]
    lane = jax.lax.broadcasted_iota(jnp.int32, (BK, S_PAD), 1)
    a2_neg = jnp.where(lane == 0, NEG, 0.0).astype(jnp.float32)

    def body(tt, carry):
        alpha, tail = carry
        e = e_scratch[tt]
        a2 = pltpu.roll(alpha, 1, 1) + a2_neg
        a3 = pltpu.roll(alpha, 2, 1) + skip_neg
        m = jnp.maximum(jnp.maximum(alpha, a2), a3)
        new = m + jnp.log(jnp.exp(alpha - m) + jnp.exp(a2 - m)
                          + jnp.exp(a3 - m)) + e
        t = pid * T_BLK + tt
        tail = jnp.where(lens == t + 1, new[:, TAIL0:], tail)
        return new, tail

    tail0 = jnp.full((BK, S_PAD - TAIL0), NEG, jnp.float32)
    alpha, tail = jax.lax.fori_loop(
        0, T_BLK, body, (alpha_ref[...], tail0))
    alpha_ref[...] = alpha
    sc = jnp.logaddexp(tail[:, S - 2 - TAIL0:S - 1 - TAIL0],
                       tail[:, S - 1 - TAIL0:S - TAIL0])
    hit = ((lens > pid * T_BLK) & (lens <= (pid + 1) * T_BLK))
    num_ref[...] = jnp.where(hit, sc, num_ref[...])


@jax.jit
def _graph_loss_impl(log_probs, log_probs_lens, word_ids, target_lengths):
    tgt = word_ids.astype(jnp.int32)
    ext = jnp.zeros((B, S), dtype=jnp.int32).at[:, 1::2].set(tgt)
    ext = jnp.concatenate(
        [ext, jnp.full((B, S_PAD - S), -1, jnp.int32)], axis=1)
    allow = jnp.concatenate(
        [jnp.zeros((B, 2), bool),
         (ext[:, 2:S] != 0) & (ext[:, 2:S] != ext[:, :S - 2])], axis=1)
    allow = jnp.concatenate(
        [allow, jnp.zeros((B, S_PAD - S), bool)], axis=1)
    skip_neg = jnp.where(allow, 0.0, NEG).astype(jnp.float32)
    lens = log_probs_lens.astype(jnp.int32).reshape(B, 1)

    num, den = pl.pallas_call(
        _fwd_kernel,
        grid=(NC, NT),
        in_specs=[
            pl.BlockSpec((BK, S_PAD), lambda c, i: (c, 0)),
            pl.BlockSpec((BK, S_PAD), lambda c, i: (c, 0)),
            pl.BlockSpec((BK, 1), lambda c, i: (c, 0)),
            pl.BlockSpec((BK, T_BLK, V), lambda c, i: (c, i, 0)),
        ],
        out_specs=[
            pl.BlockSpec((BK, 1), lambda c, i: (c, 0)),
            pl.BlockSpec((BK, 1), lambda c, i: (c, 0)),
        ],
        out_shape=[
            jax.ShapeDtypeStruct((B, 1), jnp.float32),
            jax.ShapeDtypeStruct((B, 1), jnp.float32),
        ],
        scratch_shapes=[
            pltpu.VMEM((T_BLK, BK, S_PAD), jnp.float32),
            pltpu.VMEM((BK, S_PAD), jnp.float32),
        ],
        compiler_params=pltpu.CompilerParams(
            dimension_semantics=("parallel", "arbitrary")),
    )(ext, skip_neg, lens, log_probs)

    tl = target_lengths.astype(jnp.float32)
    num_loss = -num[:, 0]
    den_loss = -den[:, 0]
    return jnp.mean(num_loss / tl) - jnp.mean(den_loss / tl)


def kernel(log_probs, log_probs_lens, word_ids, target_lengths):
    return _graph_loss_impl(log_probs, log_probs_lens, word_ids,
                            target_lengths)


# speedup vs baseline: 1.5820x; 1.5820x over previous
import functools

import jax
import jax.numpy as jnp
from jax.experimental import pallas as pl
from jax.experimental.pallas import tpu as pltpu

B, T, V, U = 16, 2048, 512, 256
S = 2 * U + 1
S_PAD = 640
TAIL0 = 384
NEG = -1e30
T_BLK = 128
NT = T // T_BLK


LOG2E = 1.4426950408889634
LN2 = 0.6931471805599453


def _fwd_kernel(ext_ref, skip_ref, lens_ref, lp_ref, num_ref, den_ref,
                e_scratch, alpha_ref, oh_scratch):
    pid = pl.program_id(0)

    @pl.when(pid == 0)
    def _init():
        lane = jax.lax.broadcasted_iota(jnp.int32, (B, S_PAD), 1)
        alpha_ref[...] = jnp.where(lane == 0, 0.0, NEG).astype(jnp.float32)
        num_ref[...] = jnp.zeros((B, 1), jnp.float32)
        den_ref[...] = jnp.zeros((B, 1), jnp.float32)
        iota_v = jax.lax.broadcasted_iota(jnp.int32, (V, S_PAD), 0)
        for b in range(B):
            oh_scratch[b] = jnp.where(ext_ref[b:b + 1, :] == iota_v,
                                      LOG2E, 0.0).astype(jnp.float32)

    row_t = (jax.lax.broadcasted_iota(jnp.int32, (T_BLK, 1), 0)
             + pid * T_BLK)
    for b in range(B):
        lp_b = lp_ref[b]
        e_scratch[:, b, :] = jnp.dot(lp_b, oh_scratch[b],
                                     preferred_element_type=jnp.float32)
        m = jnp.max(lp_b, axis=1, keepdims=True)
        lse = m + jnp.log(jnp.sum(jnp.exp(lp_b - m), axis=1, keepdims=True))
        dpart = jnp.sum(jnp.where(row_t < lens_ref[b:b + 1, 0:1], lse, 0.0),
                        axis=0, keepdims=True)
        den_ref[b:b + 1, :] = den_ref[b:b + 1, :] + dpart

    skip_neg = skip_ref[...]
    lens = lens_ref[...]
    lane = jax.lax.broadcasted_iota(jnp.int32, (B, S_PAD), 1)
    a2_neg = jnp.where(lane == 0, NEG, 0.0).astype(jnp.float32)

    def body(tt, carry):
        alpha, tail = carry
        e = e_scratch[tt]
        a2 = pltpu.roll(alpha, 1, 1) + a2_neg
        a3 = pltpu.roll(alpha, 2, 1) + skip_neg
        m = jnp.maximum(jnp.maximum(alpha, a2), a3)
        new = m + jnp.log2(jnp.exp2(alpha - m) + jnp.exp2(a2 - m)
                           + jnp.exp2(a3 - m)) + e
        t = pid * T_BLK + tt
        tail = jnp.where(lens == t + 1, new[:, TAIL0:], tail)
        return new, tail

    tail0 = jnp.full((B, S_PAD - TAIL0), NEG, jnp.float32)
    alpha, tail = jax.lax.fori_loop(
        0, T_BLK, body, (alpha_ref[...], tail0))
    alpha_ref[...] = alpha
    sc = jnp.logaddexp2(tail[:, S - 2 - TAIL0:S - 1 - TAIL0],
                        tail[:, S - 1 - TAIL0:S - TAIL0]) * LN2
    hit = ((lens > pid * T_BLK) & (lens <= (pid + 1) * T_BLK))
    num_ref[...] = jnp.where(hit, sc, num_ref[...])


@jax.jit
def _graph_loss_impl(log_probs, log_probs_lens, word_ids, target_lengths):
    tgt = word_ids.astype(jnp.int32)
    ext = jnp.zeros((B, S), dtype=jnp.int32).at[:, 1::2].set(tgt)
    ext = jnp.concatenate(
        [ext, jnp.full((B, S_PAD - S), -1, jnp.int32)], axis=1)
    allow = jnp.concatenate(
        [jnp.zeros((B, 2), bool),
         (ext[:, 2:S] != 0) & (ext[:, 2:S] != ext[:, :S - 2])], axis=1)
    allow = jnp.concatenate(
        [allow, jnp.zeros((B, S_PAD - S), bool)], axis=1)
    skip_neg = jnp.where(allow, 0.0, NEG).astype(jnp.float32)
    lens = log_probs_lens.astype(jnp.int32).reshape(B, 1)

    num, den = pl.pallas_call(
        _fwd_kernel,
        grid=(NT,),
        in_specs=[
            pl.BlockSpec((B, S_PAD), lambda i: (0, 0)),
            pl.BlockSpec((B, S_PAD), lambda i: (0, 0)),
            pl.BlockSpec((B, 1), lambda i: (0, 0)),
            pl.BlockSpec((B, T_BLK, V), lambda i: (0, i, 0)),
        ],
        out_specs=[
            pl.BlockSpec((B, 1), lambda i: (0, 0)),
            pl.BlockSpec((B, 1), lambda i: (0, 0)),
        ],
        out_shape=[
            jax.ShapeDtypeStruct((B, 1), jnp.float32),
            jax.ShapeDtypeStruct((B, 1), jnp.float32),
        ],
        scratch_shapes=[
            pltpu.VMEM((T_BLK, B, S_PAD), jnp.float32),
            pltpu.VMEM((B, S_PAD), jnp.float32),
            pltpu.VMEM((B, V, S_PAD), jnp.float32),
        ],
    )(ext, skip_neg, lens, log_probs)

    tl = target_lengths.astype(jnp.float32)
    num_loss = -num[:, 0]
    den_loss = -den[:, 0]
    return jnp.mean(num_loss / tl) - jnp.mean(den_loss / tl)


def kernel(log_probs, log_probs_lens, word_ids, target_lengths):
    return _graph_loss_impl(log_probs, log_probs_lens, word_ids,
                            target_lengths)


# speedup vs baseline: 1.8670x; 1.1802x over previous
import jax
import jax.numpy as jnp
from jax.experimental import pallas as pl
from jax.experimental.pallas import tpu as pltpu

B, T, V, U = 16, 2048, 512, 256
S = 2 * U + 1
W = 384
BLANK_LANE = 256
NEG = -1e30
T_BLK = 128
NT = T // T_BLK
LOG2E = 1.4426950408889634
LN2 = 0.6931471805599453


def _fwd_kernel(exto_ref, skip_ref, lens_ref, lp_ref, num_ref, den_ref,
                e_scratch, ao_ref, ae_ref, oh_scratch):
    pid = pl.program_id(0)

    @pl.when(pid == 0)
    def _init():
        lane = jax.lax.broadcasted_iota(jnp.int32, (B, W), 1)
        ao_ref[...] = jnp.full((B, W), NEG, jnp.float32)
        ae_ref[...] = jnp.where(lane == 0, 0.0, NEG).astype(jnp.float32)
        num_ref[...] = jnp.zeros((B, 1), jnp.float32)
        den_ref[...] = jnp.zeros((B, 1), jnp.float32)
        iota_v = jax.lax.broadcasted_iota(jnp.int32, (V, W), 0)
        for b in range(B):
            oh_scratch[b] = jnp.where(exto_ref[b:b + 1, :] == iota_v,
                                      LOG2E, 0.0).astype(jnp.float32)

    row_t = (jax.lax.broadcasted_iota(jnp.int32, (T_BLK, 1), 0)
             + pid * T_BLK)
    for b in range(B):
        lp_b = lp_ref[b]
        e_scratch[:, b, :] = jnp.dot(lp_b, oh_scratch[b],
                                     preferred_element_type=jnp.float32)
        m = jnp.max(lp_b, axis=1, keepdims=True)
        lse = m + jnp.log(jnp.sum(jnp.exp(lp_b - m), axis=1, keepdims=True))
        dpart = jnp.sum(jnp.where(row_t < lens_ref[b:b + 1, 0:1], lse, 0.0),
                        axis=0, keepdims=True)
        den_ref[b:b + 1, :] = den_ref[b:b + 1, :] + dpart

    skip_neg = skip_ref[...]
    lens = lens_ref[...]
    lane = jax.lax.broadcasted_iota(jnp.int32, (B, W), 1)
    wrap_neg = jnp.where(lane == 0, NEG, 0.0).astype(jnp.float32)

    def body(tt, carry):
        ao, ae, to, te = carry
        e = e_scratch[tt]
        eb = e[:, BLANK_LANE:BLANK_LANE + 1]
        r = pltpu.roll(ao, 1, 1)
        a3 = r + skip_neg
        mo = jnp.maximum(jnp.maximum(ao, ae), a3)
        new_ao = mo + jnp.log2(jnp.exp2(ao - mo) + jnp.exp2(ae - mo)
                               + jnp.exp2(a3 - mo)) + e
        r2 = r + wrap_neg
        me = jnp.maximum(ae, r2)
        new_ae = me + jnp.log2(jnp.exp2(ae - me) + jnp.exp2(r2 - me)) + eb
        hit = lens == pid * T_BLK + tt + 1
        to = jnp.where(hit, new_ao[:, 128:256], to)
        te = jnp.where(hit, new_ae[:, 256:384], te)
        return new_ao, new_ae, to, te

    t0 = jnp.full((B, 128), NEG, jnp.float32)
    ao, ae, to, te = jax.lax.fori_loop(
        0, T_BLK, body, (ao_ref[...], ae_ref[...], t0, t0))
    ao_ref[...] = ao
    ae_ref[...] = ae
    sc = jnp.logaddexp2(to[:, 127:128], te[:, 0:1]) * LN2
    hit_b = ((lens > pid * T_BLK) & (lens <= (pid + 1) * T_BLK))
    num_ref[...] = jnp.where(hit_b, sc, num_ref[...])


@jax.jit
def _graph_loss_impl(log_probs, log_probs_lens, word_ids, target_lengths):
    tgt = word_ids.astype(jnp.int32)
    exto = jnp.concatenate(
        [tgt, jnp.zeros((B, 1), jnp.int32),
         jnp.full((B, W - U - 1), -1, jnp.int32)], axis=1)
    allow = jnp.concatenate(
        [jnp.zeros((B, 1), bool), tgt[:, 1:] != tgt[:, :-1],
         jnp.zeros((B, W - U), bool)], axis=1)
    skip_neg = jnp.where(allow, 0.0, NEG).astype(jnp.float32)
    lens = log_probs_lens.astype(jnp.int32).reshape(B, 1)

    num, den = pl.pallas_call(
        _fwd_kernel,
        grid=(NT,),
        in_specs=[
            pl.BlockSpec((B, W), lambda i: (0, 0)),
            pl.BlockSpec((B, W), lambda i: (0, 0)),
            pl.BlockSpec((B, 1), lambda i: (0, 0)),
            pl.BlockSpec((B, T_BLK, V), lambda i: (0, i, 0)),
        ],
        out_specs=[
            pl.BlockSpec((B, 1), lambda i: (0, 0)),
            pl.BlockSpec((B, 1), lambda i: (0, 0)),
        ],
        out_shape=[
            jax.ShapeDtypeStruct((B, 1), jnp.float32),
            jax.ShapeDtypeStruct((B, 1), jnp.float32),
        ],
        scratch_shapes=[
            pltpu.VMEM((T_BLK, B, W), jnp.float32),
            pltpu.VMEM((B, W), jnp.float32),
            pltpu.VMEM((B, W), jnp.float32),
            pltpu.VMEM((B, V, W), jnp.float32),
        ],
    )(exto, skip_neg, lens, log_probs)

    tl = target_lengths.astype(jnp.float32)
    num_loss = -num[:, 0]
    den_loss = -den[:, 0]
    return jnp.mean(num_loss / tl) - jnp.mean(den_loss / tl)


def kernel(log_probs, log_probs_lens, word_ids, target_lengths):
    return _graph_loss_impl(log_probs, log_probs_lens, word_ids,
                            target_lengths)


# speedup vs baseline: 1.9602x; 1.0499x over previous
import jax
import jax.numpy as jnp
from jax.experimental import pallas as pl
from jax.experimental.pallas import tpu as pltpu

B, T, V, U = 16, 2048, 512, 256
S = 2 * U + 1
W = 384
BL = 256
NEG = -1e30
T_BLK = 128
NT = T // T_BLK
LOG2E = 1.4426950408889634
LN2 = 0.6931471805599453


def _lse(ms, terms):
    acc = jnp.exp2(terms[0] - ms)
    for t in terms[1:]:
        acc = acc + jnp.exp2(t - ms)
    return ms + jnp.log2(acc)


def _max(terms):
    m = terms[0]
    for t in terms[1:]:
        m = jnp.maximum(m, t)
    return m


def _fwd_kernel(exto_ref, exto1_ref, skip_ref, skip1_ref, lens_ref, lp_ref,
                num_ref, den_ref, e_scratch, er_scratch, ao_ref, ae_ref,
                oh_scratch, oh1_scratch):
    pid = pl.program_id(0)

    @pl.when(pid == 0)
    def _init():
        lane = jax.lax.broadcasted_iota(jnp.int32, (B, W), 1)
        ao_ref[...] = jnp.full((B, W), NEG, jnp.float32)
        ae_ref[...] = jnp.where(lane == 0, 0.0, NEG).astype(jnp.float32)
        num_ref[...] = jnp.zeros((B, 1), jnp.float32)
        den_ref[...] = jnp.zeros((B, 1), jnp.float32)
        iota_v = jax.lax.broadcasted_iota(jnp.int32, (V, W), 0)
        for b in range(B):
            oh_scratch[b] = jnp.where(exto_ref[b:b + 1, :] == iota_v,
                                      LOG2E, 0.0).astype(jnp.float32)
            oh1_scratch[b] = jnp.where(exto1_ref[b:b + 1, :] == iota_v,
                                       LOG2E, 0.0).astype(jnp.float32)

    row_t = (jax.lax.broadcasted_iota(jnp.int32, (T_BLK, 1), 0)
             + pid * T_BLK)
    for b in range(B):
        lp_b = lp_ref[b]
        e_scratch[:, b, :] = jnp.dot(lp_b, oh_scratch[b],
                                     preferred_element_type=jnp.float32)
        er_scratch[:, b, :] = jnp.dot(lp_b, oh1_scratch[b],
                                      preferred_element_type=jnp.float32)
        m = jnp.max(lp_b, axis=1, keepdims=True)
        lse = m + jnp.log(jnp.sum(jnp.exp(lp_b - m), axis=1, keepdims=True))
        dpart = jnp.sum(jnp.where(row_t < lens_ref[b:b + 1, 0:1], lse, 0.0),
                        axis=0, keepdims=True)
        den_ref[b:b + 1, :] = den_ref[b:b + 1, :] + dpart

    sk = skip_ref[...]
    sk1 = skip1_ref[...]
    lens = lens_ref[...]
    lane = jax.lax.broadcasted_iota(jnp.int32, (B, W), 1)
    wrap1 = jnp.where(lane == 0, NEG, 0.0).astype(jnp.float32)
    wrap2 = jnp.where(lane <= 1, NEG, 0.0).astype(jnp.float32)
    skc = sk[:, 128:256]

    def body(tt, carry):
        ao, ae, to, te = carry
        t0 = pid * T_BLK + 2 * tt
        e0 = e_scratch[2 * tt]
        e0r = er_scratch[2 * tt]
        e1 = e_scratch[2 * tt + 1]
        eb0 = e0[:, BL:BL + 1]
        eb1 = e1[:, BL:BL + 1]
        ao1 = pltpu.roll(ao, 1, 1) + wrap1
        ao2 = pltpu.roll(ao, 2, 1) + wrap2
        ae1 = pltpu.roll(ae, 1, 1) + wrap1

        aoc, aec = ao[:, 128:256], ae[:, 128:256]
        a3c = ao1[:, 128:256] + skc
        mc = _max([aoc, aec, a3c])
        i_to = _lse(mc, [aoc, aec, a3c]) + e0[:, 128:256]
        aoe, aee = ao1[:, 256:384], ae[:, 256:384]
        me_c = jnp.maximum(aoe, aee)
        i_te = _lse(me_c, [aoe, aee]) + eb0

        x1 = ao1 + eb0
        x2 = ao1 + e0r
        x3 = ae1 + e0r
        x4 = ao2 + e0r + sk1
        x5 = ae + eb0
        t1_ = ao + e0
        t2_ = ae + e0
        t3_ = ao1 + e0 + sk
        t6_ = x2 + sk
        t7_ = x3 + sk
        t8_ = x4 + sk
        ts = [t1_, t2_, t3_, x5, x1, t6_, t7_, t8_]
        mo = _max(ts)
        new_ao = _lse(mo, ts) + e1
        us = [x5, x1, x2, x3, x4]
        me = _max(us)
        new_ae = _lse(me, us) + eb1

        hit0 = lens == t0 + 1
        hit1 = lens == t0 + 2
        to = jnp.where(hit0, i_to, to)
        te = jnp.where(hit0, i_te, te)
        to = jnp.where(hit1, new_ao[:, 128:256], to)
        te = jnp.where(hit1, new_ae[:, 256:384], te)
        return new_ao, new_ae, to, te

    t0c = jnp.full((B, 128), NEG, jnp.float32)
    ao, ae, to, te = jax.lax.fori_loop(
        0, T_BLK // 2, body, (ao_ref[...], ae_ref[...], t0c, t0c))
    ao_ref[...] = ao
    ae_ref[...] = ae
    sc = jnp.logaddexp2(to[:, 127:128], te[:, 0:1]) * LN2
    hit_b = ((lens > pid * T_BLK) & (lens <= (pid + 1) * T_BLK))
    num_ref[...] = jnp.where(hit_b, sc, num_ref[...])


@jax.jit
def _graph_loss_impl(log_probs, log_probs_lens, word_ids, target_lengths):
    tgt = word_ids.astype(jnp.int32)
    exto = jnp.concatenate(
        [tgt, jnp.zeros((B, 1), jnp.int32),
         jnp.full((B, W - U - 1), -1, jnp.int32)], axis=1)
    exto1 = jnp.concatenate(
        [jnp.full((B, 1), -1, jnp.int32), tgt,
         jnp.full((B, W - U - 1), -1, jnp.int32)], axis=1)
    allow = jnp.concatenate(
        [jnp.zeros((B, 1), bool), tgt[:, 1:] != tgt[:, :-1],
         jnp.zeros((B, W - U), bool)], axis=1)
    skip_neg = jnp.where(allow, 0.0, NEG).astype(jnp.float32)
    skip1 = jnp.concatenate(
        [jnp.full((B, 1), NEG, jnp.float32), skip_neg[:, :-1]], axis=1)
    lens = log_probs_lens.astype(jnp.int32).reshape(B, 1)

    num, den = pl.pallas_call(
        _fwd_kernel,
        grid=(NT,),
        in_specs=[
            pl.BlockSpec((B, W), lambda i: (0, 0)),
            pl.BlockSpec((B, W), lambda i: (0, 0)),
            pl.BlockSpec((B, W), lambda i: (0, 0)),
            pl.BlockSpec((B, W), lambda i: (0, 0)),
            pl.BlockSpec((B, 1), lambda i: (0, 0)),
            pl.BlockSpec((B, T_BLK, V), lambda i: (0, i, 0)),
        ],
        out_specs=[
            pl.BlockSpec((B, 1), lambda i: (0, 0)),
            pl.BlockSpec((B, 1), lambda i: (0, 0)),
        ],
        out_shape=[
            jax.ShapeDtypeStruct((B, 1), jnp.float32),
            jax.ShapeDtypeStruct((B, 1), jnp.float32),
        ],
        scratch_shapes=[
            pltpu.VMEM((T_BLK, B, W), jnp.float32),
            pltpu.VMEM((T_BLK, B, W), jnp.float32),
            pltpu.VMEM((B, W), jnp.float32),
            pltpu.VMEM((B, W), jnp.float32),
            pltpu.VMEM((B, V, W), jnp.float32),
            pltpu.VMEM((B, V, W), jnp.float32),
        ],
    )(exto, exto1, skip_neg, skip1, lens, log_probs)

    tl = target_lengths.astype(jnp.float32)
    num_loss = -num[:, 0]
    den_loss = -den[:, 0]
    return jnp.mean(num_loss / tl) - jnp.mean(den_loss / tl)


def kernel(log_probs, log_probs_lens, word_ids, target_lengths):
    return _graph_loss_impl(log_probs, log_probs_lens, word_ids,
                            target_lengths)
